# Initial kernel scaffold; baseline (speedup 1.0000x reference)
import jax
import jax.numpy as jnp
from jax.experimental import pallas as pl


def kernel(x, edge_index, W1a, b1a, g1a, be1a, W1b, b1b, go, bo, W2a, b2a, g2a, be2a, W2b, b2b, W3, b3):
    raise NotImplementedError("write your pallas kernel here")


# speedup vs baseline: 3.2242x; 3.2242x over previous
import functools

import jax
import jax.numpy as jnp
from jax import lax
from jax.experimental import pallas as pl
from jax.experimental.pallas import tpu as pltpu
from jax.experimental.pallas import tpu_sc as plsc

_N = 10000
_LANES = 128
_GB = 128
_NT = 16
_NSC = 2
_ACC_ROWS = 10240
_EPS = 1e-5


def _make_sc_segsum(C, n_groups):
    mesh = plsc.VectorSubcoreMesh(core_axis_name="c", subcore_axis_name="s")
    passes = C // _NSC
    rows_per_tile = _ACC_ROWS // _NT

    @functools.partial(
        pl.kernel,
        out_type=jax.ShapeDtypeStruct((C, _ACC_ROWS, _LANES), jnp.float32),
        mesh=mesh,
        scratch_types=[
            pltpu.VMEM_SHARED((_ACC_ROWS, _LANES), jnp.float32),
            pltpu.VMEM((n_groups, _GB), jnp.int32),
            pltpu.VMEM((2, _GB), jnp.int32),
            pltpu.VMEM((2, _GB), jnp.int32),
            pltpu.VMEM((2, _GB, _LANES), jnp.float32),
            pltpu.SemaphoreType.DMA,
            pltpu.SemaphoreType.DMA,
        ],
    )
    def seg(xview, packed, out, acc, pk_v, sg, sd, rows, sem0, sem1):
        c = lax.axis_index("c")
        s = lax.axis_index("s")
        sems = (sem0, sem1)

        def gather_start(b):
            pltpu.async_copy(xview.at[sg.at[b]], rows.at[b], sems[b])

        def gather_wait(b):
            pltpu.make_async_copy(xview.at[sg.at[0]], rows.at[b], sems[b]).wait()

        for p in range(passes):
            chunk = c * passes + p

            def prep(g, b):
                for j in range(_GB // 16):
                    pk = pk_v[g, pl.ds(j * 16, 16)]
                    srcv = lax.shift_right_logical(pk, 14)
                    sg[b, pl.ds(j * 16, 16)] = srcv * C + chunk
                    sd[b, pl.ds(j * 16, 16)] = lax.bitwise_and(pk, 16383)

            def zb(i, carry):
                for j in range(_LANES // 16):
                    rows[0, i, pl.ds(j * 16, 16)] = jnp.zeros((16,), jnp.float32)
                return carry
            lax.fori_loop(0, _GB, zb, 0)

            def zacc(k, carry):
                pltpu.sync_copy(rows.at[0],
                                acc.at[pl.ds(s * rows_per_tile + k * _GB, _GB)])
                return carry
            lax.fori_loop(0, rows_per_tile // _GB, zacc, 0)

            pltpu.sync_copy(packed.at[s], pk_v)
            plsc.subcore_barrier()

            prep(0, 0)
            gather_start(0)

            def step(i, carry):
                g0 = i * 2
                for b in range(2):
                    g = g0 + b

                    @pl.when(g + 1 < n_groups)
                    def _():
                        prep(g + 1, 1 - b)
                        gather_start(1 - b)

                    gather_wait(b)
                    pltpu.sync_copy(rows.at[b], acc.at[sd.at[b]], add=True)
                return carry
            lax.fori_loop(0, n_groups // 2, step, 0)
            plsc.subcore_barrier()

            def wout(k, carry):
                off = s * rows_per_tile + k * _GB
                pltpu.sync_copy(acc.at[pl.ds(off, _GB)], rows.at[0])
                pltpu.sync_copy(rows.at[0], out.at[chunk, pl.ds(off, _GB)])
                return carry
            lax.fori_loop(0, rows_per_tile // _GB, wout, 0)

    return seg


def _full(shape):
    return pl.BlockSpec(shape, lambda i: (0,) * len(shape))


def _k_combine_mm_stats(x_ref, agg_ref, w_ref, b_ref, h_ref, sum_ref, sq_ref):
    C = agg_ref.shape[0]
    h = x_ref[...] + jnp.concatenate([agg_ref[k] for k in range(C)], axis=-1)
    h_ref[...] = h
    z = jnp.dot(h, w_ref[...], preferred_element_type=jnp.float32) + b_ref[...]
    zs = jnp.sum(z, axis=0, keepdims=True)
    zq = jnp.sum(z * z, axis=0, keepdims=True)

    @pl.when(pl.program_id(0) == 0)
    def _():
        sum_ref[...] = zs
        sq_ref[...] = zq

    @pl.when(pl.program_id(0) != 0)
    def _():
        sum_ref[...] += zs
        sq_ref[...] += zq


def _combine_mm_stats(x, agg, w, b, bn, f_out):
    n, f_in = x.shape
    C = agg.shape[0]
    grid = (n // bn,)
    return pl.pallas_call(
        _k_combine_mm_stats,
        grid=grid,
        in_specs=[
            pl.BlockSpec((bn, f_in), lambda i: (i, 0)),
            pl.BlockSpec((C, bn, _LANES), lambda i: (0, i, 0)),
            _full(w.shape),
            _full((1, f_out)),
        ],
        out_specs=[
            pl.BlockSpec((bn, f_in), lambda i: (i, 0)),
            _full((1, f_out)),
            _full((1, f_out)),
        ],
        out_shape=[
            jax.ShapeDtypeStruct((n, f_in), jnp.float32),
            jax.ShapeDtypeStruct((1, f_out), jnp.float32),
            jax.ShapeDtypeStruct((1, f_out), jnp.float32),
        ],
    )(x, agg, w, b.reshape(1, f_out))


def _k_mlp_stats(h_ref, w1_ref, b1_ref, s1_ref, t1_ref, w2_ref, b2_ref,
                 u_ref, sum_ref, sq_ref):
    z = jnp.dot(h_ref[...], w1_ref[...], preferred_element_type=jnp.float32) + b1_ref[...]
    a = jnp.maximum(z * s1_ref[...] + t1_ref[...], 0.0)
    z2 = jnp.dot(a, w2_ref[...], preferred_element_type=jnp.float32) + b2_ref[...]
    u = jnp.maximum(z2, 0.0)
    u_ref[...] = u
    us = jnp.sum(u, axis=0, keepdims=True)
    uq = jnp.sum(u * u, axis=0, keepdims=True)

    @pl.when(pl.program_id(0) == 0)
    def _():
        sum_ref[...] = us
        sq_ref[...] = uq

    @pl.when(pl.program_id(0) != 0)
    def _():
        sum_ref[...] += us
        sq_ref[...] += uq


def _mlp_stats(h, w1, b1, s1, t1, w2, b2, bn):
    n, f_in = h.shape
    f_mid = w1.shape[1]
    f_out = w2.shape[1]
    grid = (n // bn,)
    return pl.pallas_call(
        _k_mlp_stats,
        grid=grid,
        in_specs=[
            pl.BlockSpec((bn, f_in), lambda i: (i, 0)),
            _full(w1.shape),
            _full((1, f_mid)),
            _full((1, f_mid)),
            _full((1, f_mid)),
            _full(w2.shape),
            _full((1, f_out)),
        ],
        out_specs=[
            pl.BlockSpec((bn, f_out), lambda i: (i, 0)),
            _full((1, f_out)),
            _full((1, f_out)),
        ],
        out_shape=[
            jax.ShapeDtypeStruct((n, f_out), jnp.float32),
            jax.ShapeDtypeStruct((1, f_out), jnp.float32),
            jax.ShapeDtypeStruct((1, f_out), jnp.float32),
        ],
    )(h, w1, b1.reshape(1, f_mid), s1.reshape(1, f_mid), t1.reshape(1, f_mid),
      w2, b2.reshape(1, f_out))


def _k_scale(u_ref, s_ref, t_ref, h_ref):
    h_ref[...] = u_ref[...] * s_ref[...] + t_ref[...]


def _scale(u, s, t, bn):
    n, f = u.shape
    return pl.pallas_call(
        _k_scale,
        grid=(n // bn,),
        in_specs=[
            pl.BlockSpec((bn, f), lambda i: (i, 0)),
            _full((1, f)),
            _full((1, f)),
        ],
        out_specs=pl.BlockSpec((bn, f), lambda i: (i, 0)),
        out_shape=jax.ShapeDtypeStruct((n, f), jnp.float32),
    )(u, s.reshape(1, f), t.reshape(1, f))


def _k_mlp_out(h2_ref, w1_ref, b1_ref, s1_ref, t1_ref, w2_ref, b2_ref,
               w3_ref, b3_ref, out_ref):
    z = jnp.dot(h2_ref[...], w1_ref[...], preferred_element_type=jnp.float32) + b1_ref[...]
    a = jnp.maximum(z * s1_ref[...] + t1_ref[...], 0.0)
    z2 = jnp.dot(a, w2_ref[...], preferred_element_type=jnp.float32) + b2_ref[...]
    v = jnp.maximum(z2, 0.0)
    out_ref[...] = jnp.dot(v, w3_ref[...], preferred_element_type=jnp.float32) + b3_ref[...]


def _mlp_out(h2, w1, b1, s1, t1, w2, b2, w3, b3, bn):
    n, f_in = h2.shape
    f_mid = w1.shape[1]
    f_mid2 = w2.shape[1]
    f_out = w3.shape[1]
    return pl.pallas_call(
        _k_mlp_out,
        grid=(n // bn,),
        in_specs=[
            pl.BlockSpec((bn, f_in), lambda i: (i, 0)),
            _full(w1.shape),
            _full((1, f_mid)),
            _full((1, f_mid)),
            _full((1, f_mid)),
            _full(w2.shape),
            _full((1, f_mid2)),
            _full(w3.shape),
            _full((1, f_out)),
        ],
        out_specs=pl.BlockSpec((bn, f_out), lambda i: (i, 0)),
        out_shape=jax.ShapeDtypeStruct((n, f_out), jnp.float32),
    )(h2, w1, b1.reshape(1, f_mid), s1.reshape(1, f_mid), t1.reshape(1, f_mid),
      w2, b2.reshape(1, f_mid2), w3, b3.reshape(1, f_out))


def _bn_scale_offset(ssum, ssq, g, b, n):
    m = ssum[0] / n
    v = ssq[0] / n - m * m
    s = g * lax.rsqrt(v + _EPS)
    t = b - m * s
    return s, t


def kernel(x, edge_index, W1a, b1a, g1a, be1a, W1b, b1b, go, bo,
           W2a, b2a, g2a, be2a, W2b, b2b, W3, b3):
    n = x.shape[0]
    src = edge_index[0].astype(jnp.int32)
    dst = edge_index[1].astype(jnp.int32)
    e = src.shape[0]

    n_groups = -(-e // (_NT * _GB))
    if n_groups % 2:
        n_groups += 1
    e_pad = _NT * n_groups * _GB
    srcp = jnp.concatenate([src, jnp.zeros((e_pad - e,), jnp.int32)])
    dstp = jnp.concatenate([dst, jnp.full((e_pad - e,), n, jnp.int32)])
    packed = ((srcp << 14) | dstp).reshape(_NT, n_groups, _GB)

    bn = 2000

    agg1 = _make_sc_segsum(2, n_groups)(x.reshape(2 * n, _LANES), packed)
    h_in, zs, zq = _combine_mm_stats(x, agg1, W1a, b1a, bn, W1a.shape[1])
    s1, t1 = _bn_scale_offset(zs, zq, g1a, be1a, n)
    u, us, uq = _mlp_stats(h_in, W1a, b1a, s1, t1, W1b, b1b, bn)
    so, to = _bn_scale_offset(us, uq, go, bo, n)
    h = _scale(u, so, to, bn)

    agg2 = _make_sc_segsum(4, n_groups)(h.reshape(4 * n, _LANES), packed)
    h2, zs2, zq2 = _combine_mm_stats(h, agg2, W2a, b2a, bn, W2a.shape[1])
    s2, t2 = _bn_scale_offset(zs2, zq2, g2a, be2a, n)
    out = _mlp_out(h2, W2a, b2a, s2, t2, W2b, b2b, W3, b3, bn)
    return out
